# baseline (speedup 1.0000x reference)
import functools

import jax
import jax.numpy as jnp
from jax.experimental import pallas as pl
from jax.experimental.pallas import tpu as pltpu

B, HEADS, L, DIM, CODES = 4, 16, 4096, 64, 128
TL = 4096
HG = 4
NQ = HEADS // HG


def _onehot_half(sim, iota_f, out_ref):
    m = jnp.max(sim, axis=-1, keepdims=True)
    masked = jnp.where(sim == m, iota_f, float(CODES))
    idxf = jnp.min(masked, axis=-1, keepdims=True)
    out_ref[...] = jnp.where(iota_f == idxf, 1.0, 0.0)


def _fused_body(xt_ref, w_ref, out_ref):
    a = xt_ref[...].reshape(HG * DIM, TL)
    sim = jax.lax.dot_general(
        a, w_ref[...],
        dimension_numbers=(((0,), (0,)), ((), ())),
        preferred_element_type=jnp.float32,
    )
    iota_f = jax.lax.broadcasted_iota(
        jnp.int32, (TL, CODES), 1).astype(jnp.float32)
    for h in range(HG):
        _onehot_half(sim[:, h * CODES:(h + 1) * CODES], iota_f, out_ref.at[h])


@functools.partial(jax.jit, static_argnames=("interpret",))
def _fused_call(x, c, interpret=False):
    cT = jnp.swapaxes(c, 1, 2).reshape(NQ, HG, DIM, CODES)
    w = jnp.einsum('qkdc,kj->qkdjc', cT, jnp.eye(HG, dtype=jnp.float32))
    w = w.reshape(NQ, HG * DIM, HG * CODES)
    xt = jnp.transpose(x, (0, 1, 3, 2))
    grid = (NQ, B)
    out = pl.pallas_call(
        _fused_body,
        grid=grid,
        in_specs=[
            pl.BlockSpec((None, HG, DIM, TL), lambda q, b: (b, q, 0, 0)),
            pl.BlockSpec((None, HG * DIM, HG * CODES), lambda q, b: (q, 0, 0)),
        ],
        out_specs=pl.BlockSpec((None, HG, TL, CODES),
                               lambda q, b: (b, q, 0, 0)),
        out_shape=jax.ShapeDtypeStruct((B, HEADS, L, CODES), jnp.float32),
        compiler_params=pltpu.CompilerParams(
            dimension_semantics=("parallel", "parallel")),
        interpret=interpret,
    )(xt, w)
    return out


def kernel(x, c):
    onehot = _fused_call(x, c)
    return (onehot, c)


# speedup vs baseline: 1.2822x; 1.2822x over previous
import functools

import jax
import jax.numpy as jnp
from jax.experimental import pallas as pl
from jax.experimental.pallas import tpu as pltpu

B, HEADS, L, DIM, CODES = 4, 16, 4096, 64, 128
TL = 4096
HG = 4
NQ = HEADS // HG


def _fused_body(xt_ref, w_ref, out_ref):
    a = xt_ref[...].reshape(HG * DIM, TL)
    simr = jax.lax.dot_general(
        a, w_ref[...],
        dimension_numbers=(((0,), (0,)), ((), ())),
        preferred_element_type=jnp.float32,
    )
    iota = jax.lax.broadcasted_iota(jnp.int32, (TL, CODES), 1)
    for h in range(HG):
        sim_h = simr[:, h * CODES:(h + 1) * CODES]
        idxr = jnp.argmax(sim_h, axis=-1).astype(jnp.int32)
        idx = (CODES - 1) - idxr
        out_ref[h] = jnp.where(iota == idx[:, None], 1.0, 0.0)


@functools.partial(jax.jit, static_argnames=("interpret",))
def _fused_call(x, c, interpret=False):
    cT = jnp.swapaxes(c, 1, 2).reshape(NQ, HG, DIM, CODES)
    cT = cT[..., ::-1]
    w = jnp.einsum('qkdc,kj->qkdjc', cT, jnp.eye(HG, dtype=jnp.float32))
    w = w.reshape(NQ, HG * DIM, HG * CODES)
    xt = jnp.transpose(x, (0, 1, 3, 2))
    grid = (NQ, B)
    out = pl.pallas_call(
        _fused_body,
        grid=grid,
        in_specs=[
            pl.BlockSpec((None, HG, DIM, TL), lambda q, b: (b, q, 0, 0)),
            pl.BlockSpec((None, HG * DIM, HG * CODES), lambda q, b: (q, 0, 0)),
        ],
        out_specs=pl.BlockSpec((None, HG, TL, CODES),
                               lambda q, b: (b, q, 0, 0)),
        out_shape=jax.ShapeDtypeStruct((B, HEADS, L, CODES), jnp.float32),
        compiler_params=pltpu.CompilerParams(
            dimension_semantics=("parallel", "parallel")),
        interpret=interpret,
    )(xt, w)
    return out


def kernel(x, c):
    onehot = _fused_call(x, c)
    return (onehot, c)
